# baseline (speedup 1.0000x reference)
import functools

import jax
import jax.numpy as jnp
from jax import lax
from jax.experimental import pallas as pl
from jax.experimental.pallas import tpu as pltpu
from jax.experimental.pallas import tpu_sc as plsc

N_NODES = 10000
N_EDGES = 320000
D_IN = 128
D_HID = 64

NUM_CORES = 2
NUM_SUBCORES = 16
NUM_TILES = NUM_CORES * NUM_SUBCORES

CHUNK = 125
CH_PER_TILE = 80
NBUF = 3
LEAD = 2
N_PAD = 10240
ROWS_PER_TILE = N_PAD // NUM_SUBCORES
ZROWS = 80
TAIL_ROWS = N_NODES - 15 * ROWS_PER_TILE


@functools.lru_cache(maxsize=1)
def _make_aggregate():
  mesh = plsc.VectorSubcoreMesh(core_axis_name="c", subcore_axis_name="s",
                                num_cores=NUM_CORES,
                                num_subcores=NUM_SUBCORES)

  @functools.partial(
      pl.kernel,
      out_type=jax.ShapeDtypeStruct((NUM_CORES, N_PAD, D_HID), jnp.float32),
      mesh=mesh,
      scratch_types=[
          pltpu.VMEM((CH_PER_TILE, CHUNK), jnp.int32),
          pltpu.VMEM((CH_PER_TILE, CHUNK), jnp.int32),
          [pltpu.VMEM((CHUNK, D_HID), jnp.float32) for _ in range(NBUF)],
          pltpu.VMEM((ZROWS, D_HID), jnp.float32),
          pltpu.VMEM_SHARED((N_PAD, D_HID), jnp.float32),
          pltpu.VMEM_SHARED((N_NODES, D_HID), jnp.float32),
          [pltpu.SemaphoreType.DMA for _ in range(NBUF)],
          [pltpu.SemaphoreType.DMA for _ in range(NBUF)],
          pltpu.SemaphoreType.DMA,
      ],
      compiler_params=pltpu.CompilerParams(use_tc_tiling_on_sc=False),
  )
  def aggregate(y_hbm, src_hbm, dst_hbm, out_hbm,
                src_v, dst_v, rows, zero_v, acc, ycache, sem_g, sem_s, sem_p):
    c = lax.axis_index("c")
    s = lax.axis_index("s")
    w = c * NUM_SUBCORES + s
    row0 = s * ROWS_PER_TILE

    @pl.when(s < NUM_SUBCORES - 1)
    def _():
      pltpu.async_copy(y_hbm.at[pl.ds(row0, ROWS_PER_TILE), :],
                       ycache.at[pl.ds(row0, ROWS_PER_TILE), :], sem_p)
    @pl.when(s == NUM_SUBCORES - 1)
    def _():
      pltpu.async_copy(y_hbm.at[pl.ds(row0, TAIL_ROWS), :],
                       ycache.at[pl.ds(row0, TAIL_ROWS), :], sem_p)
    pltpu.async_copy(src_hbm.at[pl.ds(w * CH_PER_TILE, CH_PER_TILE), :],
                     src_v, sem_g[0])
    pltpu.async_copy(dst_hbm.at[pl.ds(w * CH_PER_TILE, CH_PER_TILE), :],
                     dst_v, sem_g[1])

    def zrow(r, _):
      for q in range(D_HID // 16):
        zero_v[r, pl.ds(q * 16, 16)] = jnp.zeros((16,), jnp.float32)
      return 0
    lax.fori_loop(0, ZROWS, zrow, 0)
    for k in range(ROWS_PER_TILE // ZROWS):
      pltpu.sync_copy(zero_v, acc.at[pl.ds(row0 + k * ZROWS, ZROWS), :])

    pltpu.make_async_copy(
        src_hbm.at[pl.ds(w * CH_PER_TILE, CH_PER_TILE), :],
        src_v, sem_g[0]).wait()
    pltpu.make_async_copy(
        dst_hbm.at[pl.ds(w * CH_PER_TILE, CH_PER_TILE), :],
        dst_v, sem_g[1]).wait()
    @pl.when(s < NUM_SUBCORES - 1)
    def _():
      pltpu.make_async_copy(y_hbm.at[pl.ds(row0, ROWS_PER_TILE), :],
                            ycache.at[pl.ds(row0, ROWS_PER_TILE), :],
                            sem_p).wait()
    @pl.when(s == NUM_SUBCORES - 1)
    def _():
      pltpu.make_async_copy(y_hbm.at[pl.ds(row0, TAIL_ROWS), :],
                            ycache.at[pl.ds(row0, TAIL_ROWS), :], sem_p).wait()
    plsc.subcore_barrier()

    def g_issue(j, b):
      pltpu.async_copy(ycache.at[src_v.at[j]], rows[b], sem_g[b])

    def g_wait(j, b):
      pltpu.make_async_copy(ycache.at[src_v.at[j]], rows[b], sem_g[b]).wait()

    def s_issue(j, b):
      pltpu.async_copy(rows[b], acc.at[dst_v.at[j]], sem_s[b], add=True)

    def s_wait(j, b):
      pltpu.make_async_copy(rows[b], acc.at[dst_v.at[j]], sem_s[b]).wait()

    for j in range(LEAD):
      g_issue(j, j % NBUF)

    def body(g, _):
      for b0 in range(NBUF):
        j = g * NBUF + b0
        b = b0
        bg = (b0 + LEAD) % NBUF
        @pl.when(j + LEAD < CH_PER_TILE)
        def _():
          @pl.when(j >= NBUF - LEAD)
          def _():
            s_wait(j - (NBUF - LEAD), bg)
          g_issue(j + LEAD, bg)
        g_wait(j, b)
        s_issue(j, b)
      return 0
    lax.fori_loop(0, CH_PER_TILE // NBUF, body, 0)
    for j in range((CH_PER_TILE // NBUF) * NBUF, CH_PER_TILE):
      g_wait(j, j % NBUF)
      s_issue(j, j % NBUF)
    for j in range(CH_PER_TILE - NBUF, CH_PER_TILE):
      s_wait(j, j % NBUF)

    plsc.subcore_barrier()
    pltpu.sync_copy(acc.at[pl.ds(row0, ROWS_PER_TILE), :],
                    out_hbm.at[c, pl.ds(row0, ROWS_PER_TILE), :])

  return aggregate


def _aggregate(y, src2, dst2):
  return _make_aggregate()(y, src2, dst2)


_BLK = 1000
_GRID = N_NODES // (2 * _BLK)
_NV = N_NODES // 2
_PV = N_PAD // 2


def _mm1_body(x_ref, w_ref, o_ref):
  o_ref[:, :] = jnp.dot(x_ref[:, :], w_ref[:, :],
                        preferred_element_type=jnp.float32)


def _matmul1(x, w1):
  return pl.pallas_call(
      _mm1_body,
      grid=(_GRID,),
      in_specs=[
          pl.BlockSpec((2 * _BLK, D_IN), lambda i: (i, 0)),
          pl.BlockSpec((D_IN, D_HID), lambda i: (0, 0)),
      ],
      out_specs=pl.BlockSpec((2 * _BLK, D_HID), lambda i: (i, 0)),
      out_shape=jax.ShapeDtypeStruct((N_NODES, D_HID), jnp.float32),
  )(x, w1)


def _relu_body(y_ref, pa_ref, pb_ref, b_ref, o_ref):
  o_ref[:, :] = jnp.maximum(
      y_ref[:, :] + pa_ref[0, :, :] + pb_ref[0, :, :] + b_ref[:, :], 0.0)


def _relu_sum(yv, partials, b1):
  pv = partials.reshape(NUM_CORES, _PV, 2 * D_HID)
  b1p = jnp.concatenate([b1, b1]).reshape(1, 2 * D_HID)
  return pl.pallas_call(
      _relu_body,
      grid=(_GRID,),
      in_specs=[
          pl.BlockSpec((_BLK, 2 * D_HID), lambda i: (i, 0)),
          pl.BlockSpec((1, _BLK, 2 * D_HID), lambda i: (0, i, 0)),
          pl.BlockSpec((1, _BLK, 2 * D_HID), lambda i: (1, i, 0)),
          pl.BlockSpec((1, 2 * D_HID), lambda i: (0, 0)),
      ],
      out_specs=pl.BlockSpec((_BLK, 2 * D_HID), lambda i: (i, 0)),
      out_shape=jax.ShapeDtypeStruct((_NV, 2 * D_HID), jnp.float32),
  )(yv, pv, pv, b1p)


def _out_body(h_ref, qa_ref, qb_ref, we_ref, wo_ref, b_ref, o_ref):
  g = h_ref[:, :] + qa_ref[0, :, :] + qb_ref[0, :, :]
  oe = jnp.dot(g, we_ref[:, :], preferred_element_type=jnp.float32) + b_ref[:, :]
  oo = jnp.dot(g, wo_ref[:, :], preferred_element_type=jnp.float32) + b_ref[:, :]
  st = jnp.stack([oe, oo], axis=1)
  o = st.reshape(2 * _BLK, D_IN)
  m = jnp.max(o, axis=1, keepdims=True)
  z = o - m
  o_ref[:, :] = z - jnp.log(jnp.sum(jnp.exp(z), axis=1, keepdims=True))


def _final(hv, partials, w2, b2):
  qv = partials.reshape(NUM_CORES, _PV, 2 * D_HID)
  zeros = jnp.zeros((D_HID, D_IN), jnp.float32)
  w2e = jnp.concatenate([w2, zeros], axis=0)
  w2o = jnp.concatenate([zeros, w2], axis=0)
  return pl.pallas_call(
      _out_body,
      grid=(_GRID,),
      in_specs=[
          pl.BlockSpec((_BLK, 2 * D_HID), lambda i: (i, 0)),
          pl.BlockSpec((1, _BLK, 2 * D_HID), lambda i: (0, i, 0)),
          pl.BlockSpec((1, _BLK, 2 * D_HID), lambda i: (1, i, 0)),
          pl.BlockSpec((D_IN, D_IN), lambda i: (0, 0)),
          pl.BlockSpec((D_IN, D_IN), lambda i: (0, 0)),
          pl.BlockSpec((1, D_IN), lambda i: (0, 0)),
      ],
      out_specs=pl.BlockSpec((2 * _BLK, D_IN), lambda i: (i, 0)),
      out_shape=jax.ShapeDtypeStruct((N_NODES, D_IN), jnp.float32),
  )(hv, qv, qv, w2e, w2o, b2.reshape(1, D_IN))


def kernel(x, edge_index, W1, b1, W2, b2):
  src2 = edge_index[0].reshape(NUM_TILES * CH_PER_TILE, CHUNK)
  dst2 = edge_index[1].reshape(NUM_TILES * CH_PER_TILE, CHUNK)
  y1 = _matmul1(x, W1)
  y1v = y1.reshape(_NV, 2 * D_HID)
  p1 = _aggregate(y1v.reshape(N_NODES, D_HID), src2, dst2)
  hv = _relu_sum(y1v, p1, b1)
  p2 = _aggregate(hv.reshape(N_NODES, D_HID), src2, dst2)
  return _final(hv, p2, W2, b2)


# speedup vs baseline: 1.0461x; 1.0461x over previous
import functools

import jax
import jax.numpy as jnp
from jax import lax
from jax.experimental import pallas as pl
from jax.experimental.pallas import tpu as pltpu
from jax.experimental.pallas import tpu_sc as plsc

N_NODES = 10000
N_EDGES = 320000
D_IN = 128
D_HID = 64

NUM_CORES = 2
NUM_SUBCORES = 16
NUM_TILES = NUM_CORES * NUM_SUBCORES

CHUNK = 125
CH_PER_TILE = 80
NBUF = 3
LEAD = 2
N_PAD = 10240
ROWS_PER_TILE = N_PAD // NUM_SUBCORES
ZROWS = 80
TAIL_ROWS = N_NODES - 15 * ROWS_PER_TILE


@functools.lru_cache(maxsize=1)
def _make_aggregate():
  mesh = plsc.VectorSubcoreMesh(core_axis_name="c", subcore_axis_name="s",
                                num_cores=NUM_CORES,
                                num_subcores=NUM_SUBCORES)

  @functools.partial(
      pl.kernel,
      out_type=jax.ShapeDtypeStruct((NUM_CORES, N_PAD, D_HID), jnp.float32),
      mesh=mesh,
      scratch_types=[
          pltpu.VMEM((CH_PER_TILE, CHUNK), jnp.int32),
          pltpu.VMEM((CH_PER_TILE, CHUNK), jnp.int32),
          [pltpu.VMEM((CHUNK, D_HID), jnp.float32) for _ in range(NBUF)],
          pltpu.VMEM((ZROWS, D_HID), jnp.float32),
          pltpu.VMEM_SHARED((N_PAD, D_HID), jnp.float32),
          pltpu.VMEM_SHARED((N_NODES, D_HID), jnp.float32),
          [pltpu.SemaphoreType.DMA for _ in range(NBUF)],
          [pltpu.SemaphoreType.DMA for _ in range(NBUF)],
          pltpu.SemaphoreType.DMA,
      ],
      compiler_params=pltpu.CompilerParams(use_tc_tiling_on_sc=False),
  )
  def aggregate(y_hbm, edges_hbm, out_hbm,
                src_v, dst_v, rows, zero_v, acc, ycache, sem_g, sem_s, sem_p):
    c = lax.axis_index("c")
    s = lax.axis_index("s")
    w = c * NUM_SUBCORES + s
    row0 = s * ROWS_PER_TILE

    @pl.when(s < NUM_SUBCORES - 1)
    def _():
      pltpu.async_copy(y_hbm.at[pl.ds(row0, ROWS_PER_TILE), :],
                       ycache.at[pl.ds(row0, ROWS_PER_TILE), :], sem_p)
    @pl.when(s == NUM_SUBCORES - 1)
    def _():
      pltpu.async_copy(y_hbm.at[pl.ds(row0, TAIL_ROWS), :],
                       ycache.at[pl.ds(row0, TAIL_ROWS), :], sem_p)
    pltpu.async_copy(edges_hbm.at[0, pl.ds(w * CH_PER_TILE, CH_PER_TILE), :],
                     src_v, sem_g[0])
    pltpu.async_copy(edges_hbm.at[1, pl.ds(w * CH_PER_TILE, CH_PER_TILE), :],
                     dst_v, sem_g[1])

    def zrow(r, _):
      for q in range(D_HID // 16):
        zero_v[r, pl.ds(q * 16, 16)] = jnp.zeros((16,), jnp.float32)
      return 0
    lax.fori_loop(0, ZROWS, zrow, 0)
    for k in range(ROWS_PER_TILE // ZROWS):
      pltpu.sync_copy(zero_v, acc.at[pl.ds(row0 + k * ZROWS, ZROWS), :])

    pltpu.make_async_copy(
        edges_hbm.at[0, pl.ds(w * CH_PER_TILE, CH_PER_TILE), :],
        src_v, sem_g[0]).wait()
    pltpu.make_async_copy(
        edges_hbm.at[1, pl.ds(w * CH_PER_TILE, CH_PER_TILE), :],
        dst_v, sem_g[1]).wait()
    @pl.when(s < NUM_SUBCORES - 1)
    def _():
      pltpu.make_async_copy(y_hbm.at[pl.ds(row0, ROWS_PER_TILE), :],
                            ycache.at[pl.ds(row0, ROWS_PER_TILE), :],
                            sem_p).wait()
    @pl.when(s == NUM_SUBCORES - 1)
    def _():
      pltpu.make_async_copy(y_hbm.at[pl.ds(row0, TAIL_ROWS), :],
                            ycache.at[pl.ds(row0, TAIL_ROWS), :], sem_p).wait()
    plsc.subcore_barrier()

    def g_issue(j, b):
      pltpu.async_copy(ycache.at[src_v.at[j]], rows[b], sem_g[b])

    def g_wait(j, b):
      pltpu.make_async_copy(ycache.at[src_v.at[j]], rows[b], sem_g[b]).wait()

    def s_issue(j, b):
      pltpu.async_copy(rows[b], acc.at[dst_v.at[j]], sem_s[b], add=True)

    def s_wait(j, b):
      pltpu.make_async_copy(rows[b], acc.at[dst_v.at[j]], sem_s[b]).wait()

    for j in range(LEAD):
      g_issue(j, j % NBUF)

    def body(g, _):
      for b0 in range(NBUF):
        j = g * NBUF + b0
        b = b0
        bg = (b0 + LEAD) % NBUF
        @pl.when(j + LEAD < CH_PER_TILE)
        def _():
          @pl.when(j >= NBUF - LEAD)
          def _():
            s_wait(j - (NBUF - LEAD), bg)
          g_issue(j + LEAD, bg)
        g_wait(j, b)
        s_issue(j, b)
      return 0
    lax.fori_loop(0, CH_PER_TILE // NBUF, body, 0)
    for j in range((CH_PER_TILE // NBUF) * NBUF, CH_PER_TILE):
      g_wait(j, j % NBUF)
      s_issue(j, j % NBUF)
    for j in range(CH_PER_TILE - NBUF, CH_PER_TILE):
      s_wait(j, j % NBUF)

    plsc.subcore_barrier()
    pltpu.sync_copy(acc.at[pl.ds(row0, ROWS_PER_TILE), :],
                    out_hbm.at[c, pl.ds(row0, ROWS_PER_TILE), :])

  return aggregate


def _aggregate(y, edges3):
  return _make_aggregate()(y, edges3)


_BLK = 1000
_GRID = N_NODES // (2 * _BLK)
_NV = N_NODES // 2
_PV = N_PAD // 2


def _mm1_body(x_ref, w_ref, o_ref):
  o_ref[:, :] = jnp.dot(x_ref[:, :], w_ref[:, :],
                        preferred_element_type=jnp.float32)


def _matmul1(x, w1):
  return pl.pallas_call(
      _mm1_body,
      grid=(_GRID,),
      in_specs=[
          pl.BlockSpec((2 * _BLK, D_IN), lambda i: (i, 0)),
          pl.BlockSpec((D_IN, D_HID), lambda i: (0, 0)),
      ],
      out_specs=pl.BlockSpec((2 * _BLK, D_HID), lambda i: (i, 0)),
      out_shape=jax.ShapeDtypeStruct((N_NODES, D_HID), jnp.float32),
  )(x, w1)


def _relu_body(y_ref, pa_ref, pb_ref, b_ref, o_ref):
  o_ref[:, :] = jnp.maximum(
      y_ref[:, :] + pa_ref[0, :, :] + pb_ref[0, :, :] + b_ref[:, :], 0.0)


def _relu_sum(yv, partials, b1):
  pv = partials.reshape(NUM_CORES, _PV, 2 * D_HID)
  b1p = jnp.concatenate([b1, b1]).reshape(1, 2 * D_HID)
  return pl.pallas_call(
      _relu_body,
      grid=(_GRID,),
      in_specs=[
          pl.BlockSpec((_BLK, 2 * D_HID), lambda i: (i, 0)),
          pl.BlockSpec((1, _BLK, 2 * D_HID), lambda i: (0, i, 0)),
          pl.BlockSpec((1, _BLK, 2 * D_HID), lambda i: (1, i, 0)),
          pl.BlockSpec((1, 2 * D_HID), lambda i: (0, 0)),
      ],
      out_specs=pl.BlockSpec((_BLK, 2 * D_HID), lambda i: (i, 0)),
      out_shape=jax.ShapeDtypeStruct((_NV, 2 * D_HID), jnp.float32),
  )(yv, pv, pv, b1p)


def _out_body(h_ref, qa_ref, qb_ref, we_ref, wo_ref, b_ref, o_ref):
  g = h_ref[:, :] + qa_ref[0, :, :] + qb_ref[0, :, :]
  oe = jnp.dot(g, we_ref[:, :], preferred_element_type=jnp.float32) + b_ref[:, :]
  oo = jnp.dot(g, wo_ref[:, :], preferred_element_type=jnp.float32) + b_ref[:, :]
  st = jnp.stack([oe, oo], axis=1)
  o = st.reshape(2 * _BLK, D_IN)
  m = jnp.max(o, axis=1, keepdims=True)
  z = o - m
  o_ref[:, :] = z - jnp.log(jnp.sum(jnp.exp(z), axis=1, keepdims=True))


def _final(hv, partials, w2, b2):
  qv = partials.reshape(NUM_CORES, _PV, 2 * D_HID)
  zeros = jnp.zeros((D_HID, D_IN), jnp.float32)
  w2e = jnp.concatenate([w2, zeros], axis=0)
  w2o = jnp.concatenate([zeros, w2], axis=0)
  return pl.pallas_call(
      _out_body,
      grid=(_GRID,),
      in_specs=[
          pl.BlockSpec((_BLK, 2 * D_HID), lambda i: (i, 0)),
          pl.BlockSpec((1, _BLK, 2 * D_HID), lambda i: (0, i, 0)),
          pl.BlockSpec((1, _BLK, 2 * D_HID), lambda i: (1, i, 0)),
          pl.BlockSpec((D_IN, D_IN), lambda i: (0, 0)),
          pl.BlockSpec((D_IN, D_IN), lambda i: (0, 0)),
          pl.BlockSpec((1, D_IN), lambda i: (0, 0)),
      ],
      out_specs=pl.BlockSpec((2 * _BLK, D_IN), lambda i: (i, 0)),
      out_shape=jax.ShapeDtypeStruct((N_NODES, D_IN), jnp.float32),
  )(hv, qv, qv, w2e, w2o, b2.reshape(1, D_IN))


def kernel(x, edge_index, W1, b1, W2, b2):
  edges3 = edge_index.reshape(2, NUM_TILES * CH_PER_TILE, CHUNK)
  y1 = _matmul1(x, W1)
  y1v = y1.reshape(_NV, 2 * D_HID)
  p1 = _aggregate(y1v.reshape(N_NODES, D_HID), edges3)
  hv = _relu_sum(y1v, p1, b1)
  p2 = _aggregate(hv.reshape(N_NODES, D_HID), edges3)
  return _final(hv, p2, W2, b2)
